# baseline (speedup 1.0000x reference)
import jax
import jax.numpy as jnp
from jax.experimental import pallas as pl
from jax.experimental.pallas import tpu as pltpu

_CHUNK = 4000
_N = 250
_NBUF = 8
_LOOKAHEAD = 4


def _copy_body(src, dst, buf, in_sems, out_sems):
    def in_cp(i):
        return pltpu.make_async_copy(
            src.at[pl.ds(i * _CHUNK, _CHUNK)], buf.at[i % _NBUF],
            in_sems.at[i % _NBUF])

    def out_cp(i):
        return pltpu.make_async_copy(
            buf.at[i % _NBUF], dst.at[pl.ds(i * _CHUNK, _CHUNK)],
            out_sems.at[i % _NBUF])

    for i in range(_LOOKAHEAD):
        in_cp(i).start(priority=i % 2)
    for i in range(_N):
        in_cp(i).wait()
        out_cp(i).start(priority=i % 2)
        nxt = i + _LOOKAHEAD
        if nxt < _N:
            if nxt >= _NBUF:
                out_cp(nxt - _NBUF).wait()
            in_cp(nxt).start(priority=nxt % 2)
    for i in range(max(0, _N - _NBUF), _N):
        out_cp(i).wait()


def kernel(embeddings):
    rows, dim = embeddings.shape
    return pl.pallas_call(
        _copy_body,
        out_shape=jax.ShapeDtypeStruct(embeddings.shape, embeddings.dtype),
        in_specs=[pl.BlockSpec(memory_space=pl.ANY)],
        out_specs=pl.BlockSpec(memory_space=pl.ANY),
        scratch_shapes=[
            pltpu.VMEM((_NBUF, _CHUNK, dim), embeddings.dtype),
            pltpu.SemaphoreType.DMA((_NBUF,)),
            pltpu.SemaphoreType.DMA((_NBUF,)),
        ],
    )(embeddings)


# speedup vs baseline: 1.3140x; 1.3140x over previous
import jax
import jax.numpy as jnp
from jax.experimental import pallas as pl
from jax.experimental.pallas import tpu as pltpu

_BR = 1000


def _copy_body(src_ref, dst_ref):
    dst_ref[...] = src_ref[...]


def kernel(embeddings):
    rows, dim = embeddings.shape
    v = (embeddings + jnp.float32(0.0)).reshape(8, rows // 8, dim)
    grid = (rows // 8) // _BR
    out = pl.pallas_call(
        _copy_body,
        out_shape=jax.ShapeDtypeStruct(v.shape, v.dtype),
        grid=(grid,),
        in_specs=[pl.BlockSpec((8, _BR, dim), lambda i: (0, i, 0))],
        out_specs=pl.BlockSpec((8, _BR, dim), lambda i: (0, i, 0)),
    )(v)
    return out.reshape(rows, dim) + jnp.float32(0.0)
